# baseline (speedup 1.0000x reference)
import functools

import jax
import jax.numpy as jnp
from jax import lax
from jax.experimental import pallas as pl
from jax.experimental.pallas import tpu as pltpu
from jax.experimental.pallas import tpu_sc as plsc

_B = 10000
_N = 3200000
_LANES = 16
_NW = 32
_B_PAD = 10112
_PER_W = _N // _NW
_CHUNK = 10000
_N_CHUNKS = _PER_W // _CHUNK
_SUB = _CHUNK // _LANES
_CAP = _SUB


def _sc_partial_sums(batch_idx, values):
    mesh = plsc.VectorSubcoreMesh(core_axis_name="c", subcore_axis_name="s")

    @functools.partial(
        pl.kernel,
        out_type=jax.ShapeDtypeStruct((_NW, _B_PAD), jnp.float32),
        mesh=mesh,
        scratch_types=[
            pltpu.VMEM((_CHUNK,), jnp.int32),
            pltpu.VMEM((_CHUNK,), jnp.int32),
            pltpu.VMEM((_CHUNK,), jnp.float32),
            pltpu.VMEM((_CHUNK,), jnp.float32),
            pltpu.VMEM((_B_PAD,), jnp.float32),
            pltpu.VMEM((_LANES * _CAP,), jnp.int32),
            pltpu.VMEM((_LANES * _CAP,), jnp.float32),
            pltpu.SemaphoreType.DMA,
            pltpu.SemaphoreType.DMA,
        ],
        compiler_params=pltpu.CompilerParams(needs_layout_passes=False),
    )
    def k(idx_hbm, val_hbm, out_hbm, idx0, idx1, val0, val1, acc,
          log_seg, log_sum, sem0, sem1):
        wid = lax.axis_index("s") * 2 + lax.axis_index("c")
        base = wid * _PER_W
        idx_bufs, val_bufs, sems = (idx0, idx1), (val0, val1), (sem0, sem1)
        lanebase = lax.iota(jnp.int32, _LANES) * _SUB
        lanecap = lax.iota(jnp.int32, _LANES) * _CAP

        def zero_body(i, _):
            for u in range(8):
                acc[pl.ds((i * 8 + u) * _LANES, _LANES)] = jnp.zeros(
                    (_LANES,), jnp.float32)
            return 0

        lax.fori_loop(0, _B_PAD // (8 * _LANES), zero_body, 0)

        def start(c):
            off = base + c * _CHUNK
            b = c & 1
            ci = pltpu.async_copy(
                idx_hbm.at[pl.ds(off, _CHUNK)], idx_bufs[b], sems[b])
            cv = pltpu.async_copy(
                val_hbm.at[pl.ds(off, _CHUNK)], val_bufs[b], sems[b])
            return ci, cv

        pending = {0: start(0)}
        for c in range(_N_CHUNKS):
            b = c & 1
            ci, cv = pending.pop(b)
            ci.wait()
            cv.wait()
            if c + 1 < _N_CHUNKS:
                pending[1 - b] = start(c + 1)
            ib, vb = idx_bufs[b], val_bufs[b]

            prev = plsc.load_gather(ib, [lanebase])
            run = plsc.load_gather(vb, [lanebase])

            def step(t, carry, ib=ib, vb=vb):
                prev, run, cur = carry
                ixs = lanebase + t
                iv = plsc.load_gather(ib, [ixs])
                vv = plsc.load_gather(vb, [ixs])
                neq = iv != prev
                plsc.store_scatter(log_seg, [cur], prev, mask=neq)
                plsc.store_scatter(log_sum, [cur], run, mask=neq)
                cur = cur + neq.astype(jnp.int32)
                run = jnp.where(neq, vv, run + vv)
                return iv, run, cur

            prev, run, cur = lax.fori_loop(
                1, _SUB, step, (prev, run, lanecap), unroll=13)
            plsc.addupdate_scatter(acc, [prev], run)

            counts = cur - lanecap
            maxc = jnp.max(counts)

            def replay(j, _):
                pos = lanecap + j
                seg = plsc.load_gather(log_seg, [pos])
                val = plsc.load_gather(log_sum, [pos])
                plsc.addupdate_scatter(acc, [seg], val, mask=j < counts)
                return 0

            lax.fori_loop(0, maxc, replay, 0)
        pltpu.sync_copy(acc, out_hbm.at[wid])

    return k(batch_idx, values)


def _combine(tensor, partials):
    def body(t_ref, p_ref, o_ref):
        refs = jnp.sum(p_ref[...], axis=0)
        o_ref[...] = t_ref[...] - refs[:_B]

    return pl.pallas_call(
        body,
        out_shape=jax.ShapeDtypeStruct((_B,), jnp.float32),
    )(tensor, partials)


def kernel(tensor, batch_idx, atomic_numbers, element_references):
    vals = atomic_numbers.astype(jnp.float32)
    partials = _sc_partial_sums(batch_idx, vals)
    return _combine(tensor, partials)


# speedup vs baseline: 1.7915x; 1.7915x over previous
import functools

import jax
import jax.numpy as jnp
from jax import lax
from jax.experimental import pallas as pl
from jax.experimental.pallas import tpu as pltpu
from jax.experimental.pallas import tpu_sc as plsc

_B = 10000
_N = 3200000
_LANES = 16
_NW = 32
_B_PAD = 10112
_PER_W = _N // _NW
_CHUNK = 10000
_N_CHUNKS = _PER_W // _CHUNK
_SUB = _CHUNK // _LANES
_CAP = _SUB


def _sc_partial_sums(batch_idx, values):
    mesh = plsc.VectorSubcoreMesh(core_axis_name="c", subcore_axis_name="s")

    @functools.partial(
        pl.kernel,
        out_type=jax.ShapeDtypeStruct((_NW, _B_PAD), jnp.float32),
        mesh=mesh,
        scratch_types=[
            pltpu.VMEM((_CHUNK,), jnp.int32),
            pltpu.VMEM((_CHUNK,), jnp.int32),
            pltpu.VMEM((_CHUNK,), jnp.float32),
            pltpu.VMEM((_CHUNK,), jnp.float32),
            pltpu.VMEM((_B_PAD,), jnp.float32),
            pltpu.VMEM((_LANES * _CAP,), jnp.int32),
            pltpu.VMEM((_LANES * _CAP,), jnp.float32),
            pltpu.SemaphoreType.DMA,
            pltpu.SemaphoreType.DMA,
        ],
        compiler_params=pltpu.CompilerParams(needs_layout_passes=False),
    )
    def k(idx_hbm, val_hbm, out_hbm, idx0, idx1, val0, val1, acc,
          log_seg, log_sum, sem0, sem1):
        wid = lax.axis_index("s") * 2 + lax.axis_index("c")
        base = wid * _PER_W
        idx_bufs, val_bufs, sems = (idx0, idx1), (val0, val1), (sem0, sem1)
        lanebase = lax.iota(jnp.int32, _LANES) * _SUB
        lanecap = lax.iota(jnp.int32, _LANES) * _CAP

        def zero_body(i, _):
            for u in range(8):
                acc[pl.ds((i * 8 + u) * _LANES, _LANES)] = jnp.zeros(
                    (_LANES,), jnp.float32)
            return 0

        lax.fori_loop(0, _B_PAD // (8 * _LANES), zero_body, 0)

        def start(c):
            off = base + c * _CHUNK
            b = c & 1
            ci = pltpu.async_copy(
                idx_hbm.at[pl.ds(off, _CHUNK)], idx_bufs[b], sems[b])
            cv = pltpu.async_copy(
                val_hbm.at[pl.ds(off, _CHUNK)], val_bufs[b], sems[b])
            return ci, cv

        pending = {0: start(0)}
        for c in range(_N_CHUNKS):
            b = c & 1
            ci, cv = pending.pop(b)
            ci.wait()
            cv.wait()
            if c + 1 < _N_CHUNKS:
                pending[1 - b] = start(c + 1)
            ib, vb = idx_bufs[b], val_bufs[b]

            prev = plsc.load_gather(ib, [lanebase])
            run = plsc.load_gather(vb, [lanebase])

            @plsc.parallel_loop(1, _SUB, unroll=13,
                                carry=(prev, run, lanecap))
            def step(t, carry, ib=ib, vb=vb):
                prev, run, cur = carry
                ixs = lanebase + t
                iv = plsc.load_gather(ib, [ixs])
                vv = plsc.load_gather(vb, [ixs])
                neq = iv != prev
                plsc.store_scatter(log_seg, [cur], prev, mask=neq)
                plsc.store_scatter(log_sum, [cur], run, mask=neq)
                cur = cur + neq.astype(jnp.int32)
                run = jnp.where(neq, vv, run + vv)
                return iv, run, cur

            prev, run, cur = step
            plsc.addupdate_scatter(acc, [prev], run)

            counts = cur - lanecap
            maxc = jnp.max(counts)

            def replay(j, _):
                pos = lanecap + j
                seg = plsc.load_gather(log_seg, [pos])
                val = plsc.load_gather(log_sum, [pos])
                plsc.addupdate_scatter(acc, [seg], val, mask=j < counts)
                return 0

            lax.fori_loop(0, maxc, replay, 0)
        pltpu.sync_copy(acc, out_hbm.at[wid])

    return k(batch_idx, values)


def _combine(tensor, partials):
    def body(t_ref, p_ref, o_ref):
        refs = jnp.sum(p_ref[...], axis=0)
        o_ref[...] = t_ref[...] - refs[:_B]

    return pl.pallas_call(
        body,
        out_shape=jax.ShapeDtypeStruct((_B,), jnp.float32),
    )(tensor, partials)


def kernel(tensor, batch_idx, atomic_numbers, element_references):
    vals = atomic_numbers.astype(jnp.float32)
    partials = _sc_partial_sums(batch_idx, vals)
    return _combine(tensor, partials)
